# baseline (speedup 1.0000x reference)
import jax
import jax.numpy as jnp
from jax import lax
from jax.experimental import pallas as pl
from jax.experimental.pallas import tpu as pltpu
from jax.experimental.pallas import tpu_sc as plsc

_N = 20000
_C = 10
_NMS_THR = 0.5
_SCORE_THR = 0.05
_MAX_BOXES = 100

_NB = _N // 16
_NSB = 79
_L1 = _NSB * 16
_NL2 = 5
_KPAD = 112
_NCORES = 2
_NSUBCORES = 16


def _tree_max(vs):
    while len(vs) > 1:
        vs = [jnp.maximum(a, b) for a, b in zip(vs[::2], vs[1::2])] + (
            [vs[-1]] if len(vs) % 2 else [])
    return vs[0]


def _sc_body(coords_hbm, scores_hbm, out_hbm,
             x1v, y1v, x2v, y2v, sv, l1, l2s,
             kx1, ky1, kx2, ky2, karea, kidx, kval, sem_s, sem_c):
    cid = lax.axis_index("c")
    sid = lax.axis_index("s")
    wid = sid * _NCORES + cid

    @pl.when(wid < _C)
    def _work():
        iota = lax.iota(jnp.int32, 16)
        iota16 = iota * 16
        lane0 = iota == 0
        neg1 = jnp.full((16,), -1.0, jnp.float32)
        big = jnp.int32(1 << 30)

        cp_s = pltpu.async_copy(scores_hbm.at[pl.ds(wid * _N, _N)], sv, sem_s)
        cps = [pltpu.async_copy(coords_hbm.at[pl.ds(j * _N, _N)], dst, sem_c)
               for j, dst in enumerate((x1v, y1v, x2v, y2v))]
        cp_s.wait()

        for j in range(_KPAD // 16):
            kidx[pl.ds(j * 16, 16)] = jnp.zeros((16,), jnp.int32)

        def build_l1(g, c):
            base = g * 256
            vs = []
            for k in range(16):
                v = plsc.load_gather(sv, [base + iota16 + k])
                vs.append(jnp.where(v > _SCORE_THR, v, -1.0))
            l1[pl.ds(g * 16, 16)] = _tree_max(vs)
            return c

        lax.fori_loop(0, _NSB - 1, build_l1, 0)
        base = (_NSB - 1) * 256
        vs = []
        for k in range(16):
            ids = base + iota16 + k
            v = plsc.load_gather(sv, [jnp.minimum(ids, _N - 1)])
            vs.append(jnp.where((v > _SCORE_THR) & (ids < _N), v, -1.0))
        l1[pl.ds((_NSB - 1) * 16, 16)] = _tree_max(vs)

        def build_l2(g2, c):
            vs = []
            for k in range(16):
                ids = g2 * 256 + iota16 + k
                v = plsc.load_gather(l1, [jnp.minimum(ids, _L1 - 1)])
                vs.append(jnp.where(ids < _L1, v, -1.0))
            l2s[pl.ds(g2 * 16, 16)] = _tree_max(vs)
            return c

        lax.fori_loop(0, _NL2, build_l2, 0)
        l2 = [l2s[pl.ds(j * 16, 16)] for j in range(_NL2)]
        gmax0 = jnp.max(_tree_max(list(l2)))

        for cp in cps:
            cp.wait()

        def cond(carry):
            kept, gmax = carry[0], carry[1]
            return (kept < _MAX_BOXES) & (gmax > 0.0)

        def body(carry):
            kept, gmax = carry[0], carry[1]
            l2 = list(carry[2:])
            gmaxv = jnp.broadcast_to(gmax, (16,))
            keptv = jnp.broadcast_to(kept, (16,))

            sbv = jnp.full((16,), big, jnp.int32)
            for j in range(_NL2):
                f = plsc.all_reduce_ffs(l2[j] == gmaxv)
                sbv = jnp.minimum(sbv, jnp.where(f < 16, f + j * 16, big))
            v1 = plsc.load_gather(l1, [sbv * 16 + iota])
            fb = plsc.all_reduce_ffs(v1 == gmaxv)
            blkv = sbv * 16 + fb
            v0r = plsc.load_gather(sv, [blkv * 16 + iota])
            v0 = jnp.where(v0r > _SCORE_THR, v0r, -1.0)
            fl = plsc.all_reduce_ffs(v0 == gmaxv)
            idxv = blkv * 16 + fl

            bx1 = plsc.load_gather(x1v, [idxv])
            by1 = plsc.load_gather(y1v, [idxv])
            bx2 = plsc.load_gather(x2v, [idxv])
            by2 = plsc.load_gather(y2v, [idxv])
            area_c = (jnp.maximum(bx2 - bx1, 0.0)
                      * jnp.maximum(by2 - by1, 0.0))

            def iou_blk(j, bacc):
                ox1 = kx1[pl.ds(j * 16, 16)]
                oy1 = ky1[pl.ds(j * 16, 16)]
                ox2 = kx2[pl.ds(j * 16, 16)]
                oy2 = ky2[pl.ds(j * 16, 16)]
                ix1 = jnp.maximum(bx1, ox1)
                iy1 = jnp.maximum(by1, oy1)
                ix2 = jnp.minimum(bx2, ox2)
                iy2 = jnp.minimum(by2, oy2)
                inter = (jnp.maximum(ix2 - ix1, 0.0)
                         * jnp.maximum(iy2 - iy1, 0.0))
                areak = karea[pl.ds(j * 16, 16)]
                iou = inter / (area_c + areak - inter + 1e-8)
                return bacc | ((iou > _NMS_THR) & ((iota + j * 16) < keptv))

            nblk = (kept + 15) >> 4
            badacc = lax.fori_loop(0, nblk, iou_blk,
                                   jnp.zeros((16,), jnp.bool_))
            sup = jnp.max(jnp.where(badacc, 1, 0))

            accmask = lane0 & jnp.broadcast_to(sup == 0, (16,))
            plsc.store_scatter(kx1, [keptv], bx1, mask=accmask)
            plsc.store_scatter(ky1, [keptv], by1, mask=accmask)
            plsc.store_scatter(kx2, [keptv], bx2, mask=accmask)
            plsc.store_scatter(ky2, [keptv], by2, mask=accmask)
            plsc.store_scatter(karea, [keptv], area_c, mask=accmask)
            plsc.store_scatter(kidx, [keptv], idxv, mask=accmask)
            plsc.store_scatter(kval, [keptv], gmaxv, mask=accmask)

            plsc.store_scatter(sv, [idxv], neg1, mask=lane0)
            m0 = jnp.max(jnp.where(iota == fl, -1.0, v0))
            r1 = jnp.max(jnp.where(iota == fb, -1.0, v1))
            l2ex = [jnp.where(iota + j * 16 == sbv, -1.0, l2[j])
                    for j in range(_NL2)]
            r2 = jnp.max(_tree_max(l2ex))
            m1 = jnp.maximum(r1, m0)
            gmax_new = jnp.maximum(r2, m1)
            plsc.store_scatter(l1, [blkv], jnp.broadcast_to(m0, (16,)),
                               mask=lane0)
            m1v = jnp.broadcast_to(m1, (16,))
            l2n = [jnp.where(iota + j * 16 == sbv, m1v, l2[j])
                   for j in range(_NL2)]
            kept_new = kept + jnp.where(sup == 0, 1, 0)
            return (kept_new, gmax_new, *l2n)

        fin = lax.while_loop(cond, body, (jnp.int32(0), gmax0, *l2))
        kept_f = fin[0]

        zeros16 = jnp.zeros((16,), jnp.float32)

        def zero_blk(b, c):
            for k in range(8):
                sv[pl.ds(b * 128 + k * 16, 16)] = zeros16
            return c

        lax.fori_loop(0, _N // 128, zero_blk, 0)
        for k in range(2):
            sv[pl.ds(19968 + k * 16, 16)] = zeros16

        for j in range(_KPAD // 16):
            mask = (iota + j * 16) < kept_f
            plsc.store_scatter(sv, [kidx[pl.ds(j * 16, 16)]],
                               kval[pl.ds(j * 16, 16)], mask=mask)
        pltpu.sync_copy(sv, out_hbm.at[pl.ds(wid * _N, _N)])


@jax.jit
def _nms_sc(coords_flat, scores_flat):
    mesh = plsc.VectorSubcoreMesh(core_axis_name="c", subcore_axis_name="s",
                                  num_cores=_NCORES, num_subcores=_NSUBCORES)
    run = pl.kernel(
        _sc_body,
        out_type=jax.ShapeDtypeStruct((_C * _N,), jnp.float32),
        mesh=mesh,
        compiler_params=pltpu.CompilerParams(needs_layout_passes=False),
        scratch_types=[
            pltpu.VMEM((_N,), jnp.float32),
            pltpu.VMEM((_N,), jnp.float32),
            pltpu.VMEM((_N,), jnp.float32),
            pltpu.VMEM((_N,), jnp.float32),
            pltpu.VMEM((_N,), jnp.float32),
            pltpu.VMEM((_L1,), jnp.float32),
            pltpu.VMEM((_NL2 * 16,), jnp.float32),
            pltpu.VMEM((_KPAD,), jnp.float32),
            pltpu.VMEM((_KPAD,), jnp.float32),
            pltpu.VMEM((_KPAD,), jnp.float32),
            pltpu.VMEM((_KPAD,), jnp.float32),
            pltpu.VMEM((_KPAD,), jnp.float32),
            pltpu.VMEM((_KPAD,), jnp.int32),
            pltpu.VMEM((_KPAD,), jnp.float32),
            pltpu.SemaphoreType.DMA,
            pltpu.SemaphoreType.DMA,
        ],
    )
    return run(coords_flat, scores_flat)


def kernel(boxes, classification):
    boxes0 = boxes[0]
    cls0 = classification[0]
    coords_flat = boxes0.T.reshape(-1)
    scores_flat = cls0.T.reshape(-1)
    selected = _nms_sc(coords_flat, scores_flat).reshape(_C, _N)
    detections = jnp.concatenate([boxes0, selected.T], axis=1)
    return detections[None, ...]


# speedup vs baseline: 1.0390x; 1.0390x over previous
import jax
import jax.numpy as jnp
from jax import lax
from jax.experimental import pallas as pl
from jax.experimental.pallas import tpu as pltpu
from jax.experimental.pallas import tpu_sc as plsc

_N = 20000
_C = 10
_NMS_THR = 0.5
_SCORE_THR = 0.05
_MAX_BOXES = 100

_NB = _N // 16
_NSB = 79
_L1 = _NSB * 16
_NL2 = 5
_KPAD = 112
_NCORES = 1
_NSUBCORES = 16


def _tree_max(vs):
    while len(vs) > 1:
        vs = [jnp.maximum(a, b) for a, b in zip(vs[::2], vs[1::2])] + (
            [vs[-1]] if len(vs) % 2 else [])
    return vs[0]


def _sc_body(coords_hbm, scores_hbm, out_hbm,
             x1v, y1v, x2v, y2v, sv, l1, l2s,
             kx1, ky1, kx2, ky2, karea, kidx, kval, sem_s, sem_c):
    cid = lax.axis_index("c")
    sid = lax.axis_index("s")
    wid = cid * _NSUBCORES + sid

    @pl.when(wid < _C)
    def _work():
        iota = lax.iota(jnp.int32, 16)
        iota16 = iota * 16
        lane0 = iota == 0
        neg1 = jnp.full((16,), -1.0, jnp.float32)
        big = jnp.int32(1 << 30)

        cp_s = pltpu.async_copy(scores_hbm.at[pl.ds(wid * _N, _N)], sv, sem_s)
        cps = [pltpu.async_copy(coords_hbm.at[pl.ds(j * _N, _N)], dst, sem_c)
               for j, dst in enumerate((x1v, y1v, x2v, y2v))]
        cp_s.wait()

        for j in range(_KPAD // 16):
            kidx[pl.ds(j * 16, 16)] = jnp.zeros((16,), jnp.int32)

        def build_l1(g, c):
            base = g * 256
            vs = []
            for k in range(16):
                v = plsc.load_gather(sv, [base + iota16 + k])
                vs.append(jnp.where(v > _SCORE_THR, v, -1.0))
            l1[pl.ds(g * 16, 16)] = _tree_max(vs)
            return c

        lax.fori_loop(0, _NSB - 1, build_l1, 0)
        base = (_NSB - 1) * 256
        vs = []
        for k in range(16):
            ids = base + iota16 + k
            v = plsc.load_gather(sv, [jnp.minimum(ids, _N - 1)])
            vs.append(jnp.where((v > _SCORE_THR) & (ids < _N), v, -1.0))
        l1[pl.ds((_NSB - 1) * 16, 16)] = _tree_max(vs)

        def build_l2(g2, c):
            vs = []
            for k in range(16):
                ids = g2 * 256 + iota16 + k
                v = plsc.load_gather(l1, [jnp.minimum(ids, _L1 - 1)])
                vs.append(jnp.where(ids < _L1, v, -1.0))
            l2s[pl.ds(g2 * 16, 16)] = _tree_max(vs)
            return c

        lax.fori_loop(0, _NL2, build_l2, 0)
        l2 = [l2s[pl.ds(j * 16, 16)] for j in range(_NL2)]
        gmax0 = jnp.max(_tree_max(list(l2)))

        for cp in cps:
            cp.wait()

        def cond(carry):
            kept, gmax = carry[0], carry[1]
            return (kept < _MAX_BOXES) & (gmax > 0.0)

        def body(carry):
            kept, gmax = carry[0], carry[1]
            l2 = list(carry[2:])
            gmaxv = jnp.broadcast_to(gmax, (16,))
            keptv = jnp.broadcast_to(kept, (16,))

            sbv = jnp.full((16,), big, jnp.int32)
            for j in range(_NL2):
                f = plsc.all_reduce_ffs(l2[j] == gmaxv)
                sbv = jnp.minimum(sbv, jnp.where(f < 16, f + j * 16, big))
            v1 = plsc.load_gather(l1, [sbv * 16 + iota])
            fb = plsc.all_reduce_ffs(v1 == gmaxv)
            blkv = sbv * 16 + fb
            v0r = plsc.load_gather(sv, [blkv * 16 + iota])
            v0 = jnp.where(v0r > _SCORE_THR, v0r, -1.0)
            fl = plsc.all_reduce_ffs(v0 == gmaxv)
            idxv = blkv * 16 + fl

            bx1 = plsc.load_gather(x1v, [idxv])
            by1 = plsc.load_gather(y1v, [idxv])
            bx2 = plsc.load_gather(x2v, [idxv])
            by2 = plsc.load_gather(y2v, [idxv])
            area_c = (jnp.maximum(bx2 - bx1, 0.0)
                      * jnp.maximum(by2 - by1, 0.0))

            def iou_blk(j, bacc):
                ox1 = kx1[pl.ds(j * 16, 16)]
                oy1 = ky1[pl.ds(j * 16, 16)]
                ox2 = kx2[pl.ds(j * 16, 16)]
                oy2 = ky2[pl.ds(j * 16, 16)]
                ix1 = jnp.maximum(bx1, ox1)
                iy1 = jnp.maximum(by1, oy1)
                ix2 = jnp.minimum(bx2, ox2)
                iy2 = jnp.minimum(by2, oy2)
                inter = (jnp.maximum(ix2 - ix1, 0.0)
                         * jnp.maximum(iy2 - iy1, 0.0))
                areak = karea[pl.ds(j * 16, 16)]
                iou = inter / (area_c + areak - inter + 1e-8)
                return bacc | ((iou > _NMS_THR) & ((iota + j * 16) < keptv))

            nblk = (kept + 15) >> 4
            badacc = lax.fori_loop(0, nblk, iou_blk,
                                   jnp.zeros((16,), jnp.bool_))
            sup = jnp.max(jnp.where(badacc, 1, 0))

            accmask = lane0 & jnp.broadcast_to(sup == 0, (16,))
            plsc.store_scatter(kx1, [keptv], bx1, mask=accmask)
            plsc.store_scatter(ky1, [keptv], by1, mask=accmask)
            plsc.store_scatter(kx2, [keptv], bx2, mask=accmask)
            plsc.store_scatter(ky2, [keptv], by2, mask=accmask)
            plsc.store_scatter(karea, [keptv], area_c, mask=accmask)
            plsc.store_scatter(kidx, [keptv], idxv, mask=accmask)
            plsc.store_scatter(kval, [keptv], gmaxv, mask=accmask)

            plsc.store_scatter(sv, [idxv], neg1, mask=lane0)
            m0 = jnp.max(jnp.where(iota == fl, -1.0, v0))
            r1 = jnp.max(jnp.where(iota == fb, -1.0, v1))
            l2ex = [jnp.where(iota + j * 16 == sbv, -1.0, l2[j])
                    for j in range(_NL2)]
            r2 = jnp.max(_tree_max(l2ex))
            m1 = jnp.maximum(r1, m0)
            gmax_new = jnp.maximum(r2, m1)
            plsc.store_scatter(l1, [blkv], jnp.broadcast_to(m0, (16,)),
                               mask=lane0)
            m1v = jnp.broadcast_to(m1, (16,))
            l2n = [jnp.where(iota + j * 16 == sbv, m1v, l2[j])
                   for j in range(_NL2)]
            kept_new = kept + jnp.where(sup == 0, 1, 0)
            return (kept_new, gmax_new, *l2n)

        fin = lax.while_loop(cond, body, (jnp.int32(0), gmax0, *l2))
        kept_f = fin[0]

        zeros16 = jnp.zeros((16,), jnp.float32)

        def zero_blk(b, c):
            for k in range(8):
                sv[pl.ds(b * 128 + k * 16, 16)] = zeros16
            return c

        lax.fori_loop(0, _N // 128, zero_blk, 0)
        for k in range(2):
            sv[pl.ds(19968 + k * 16, 16)] = zeros16

        for j in range(_KPAD // 16):
            mask = (iota + j * 16) < kept_f
            plsc.store_scatter(sv, [kidx[pl.ds(j * 16, 16)]],
                               kval[pl.ds(j * 16, 16)], mask=mask)
        pltpu.sync_copy(sv, out_hbm.at[pl.ds(wid * _N, _N)])


@jax.jit
def _nms_sc(coords_flat, scores_flat):
    mesh = plsc.VectorSubcoreMesh(core_axis_name="c", subcore_axis_name="s",
                                  num_cores=_NCORES, num_subcores=_NSUBCORES)
    run = pl.kernel(
        _sc_body,
        out_type=jax.ShapeDtypeStruct((_C * _N,), jnp.float32),
        mesh=mesh,
        compiler_params=pltpu.CompilerParams(needs_layout_passes=False),
        scratch_types=[
            pltpu.VMEM((_N,), jnp.float32),
            pltpu.VMEM((_N,), jnp.float32),
            pltpu.VMEM((_N,), jnp.float32),
            pltpu.VMEM((_N,), jnp.float32),
            pltpu.VMEM((_N,), jnp.float32),
            pltpu.VMEM((_L1,), jnp.float32),
            pltpu.VMEM((_NL2 * 16,), jnp.float32),
            pltpu.VMEM((_KPAD,), jnp.float32),
            pltpu.VMEM((_KPAD,), jnp.float32),
            pltpu.VMEM((_KPAD,), jnp.float32),
            pltpu.VMEM((_KPAD,), jnp.float32),
            pltpu.VMEM((_KPAD,), jnp.float32),
            pltpu.VMEM((_KPAD,), jnp.int32),
            pltpu.VMEM((_KPAD,), jnp.float32),
            pltpu.SemaphoreType.DMA,
            pltpu.SemaphoreType.DMA,
        ],
    )
    return run(coords_flat, scores_flat)


def kernel(boxes, classification):
    boxes0 = boxes[0]
    cls0 = classification[0]
    coords_flat = boxes0.T.reshape(-1)
    scores_flat = cls0.T.reshape(-1)
    selected = _nms_sc(coords_flat, scores_flat).reshape(_C, _N)
    detections = jnp.concatenate([boxes0, selected.T], axis=1)
    return detections[None, ...]
